# baseline (speedup 1.0000x reference)
import jax
import jax.numpy as jnp
from jax import lax
from jax.experimental import pallas as pl
from jax.experimental.pallas import tpu as pltpu

V0, V1, V2 = 3584, 1536, 512
E = 128
B = 1024
LD, LP = 32, 16

V0P, V1P, V2P = 3584, 1536, 512
W = 1024
R = 6040
OFF_EMB0 = 0
OFF_EMB1 = 3584
OFF_WH = 5120
OFF_BH = 5376
OFF_W1 = 5384
OFF_B1 = 5512
OFF_W2 = 5520
OFF_B2 = 6032

TB = 256
CONTRACT0 = (((0,), (0,)), ((), ()))


def _body(didx_ref, pidx_ref, p_ref, out_ref, cnt_ref, idxt_ref):
    f32 = jnp.float32
    bf16 = jnp.bfloat16
    i16 = jnp.int16

    def bag(idx_ref, ntaps, vocab_p, emb_off):
        idxt_ref[:ntaps, :] = jnp.transpose(idx_ref[...], (1, 0))
        row = lax.broadcasted_iota(i16, (vocab_p, TB), 0)
        cnt_ref[:vocab_p, :] = jnp.zeros((vocab_p, TB), i16)

        def tap8(i, c):
            v8 = idxt_ref[pl.ds(i * 8, 8), :].astype(i16)
            m = (row == v8[0:1, :]).astype(i16)
            for j in range(1, 8):
                m = m + (row == v8[j:j + 1, :]).astype(i16)
            cnt_ref[:vocab_p, :] = cnt_ref[:vocab_p, :] + m
            return c

        lax.fori_loop(0, ntaps // 8, tap8, 0)
        emb = p_ref[emb_off:emb_off + vocab_p, :E].astype(bf16)
        return lax.dot_general(cnt_ref[:vocab_p, :].astype(bf16), emb,
                               CONTRACT0, preferred_element_type=f32)

    dsum = bag(didx_ref, LD, V0P, OFF_EMB0)
    psum = bag(pidx_ref, LP, V1P, OFF_EMB1)
    hr = jnp.concatenate([dsum, psum], axis=1).astype(bf16)

    wh = p_ref[OFF_WH:OFF_WH + 2 * E, :E].astype(bf16)
    bh = p_ref[OFF_BH:OFF_BH + 1, :E]
    rep = jnp.dot(hr, wh, preferred_element_type=f32) + bh

    w1 = p_ref[OFF_W1:OFF_W1 + E, :8 * E].astype(bf16)
    b1 = p_ref[OFF_B1:OFF_B1 + 1, :8 * E]
    h = jnp.maximum(jnp.dot(rep.astype(bf16), w1,
                            preferred_element_type=f32) + b1,
                    0.0).astype(bf16)

    w2a = p_ref[OFF_W2:OFF_W2 + 4 * E, 0:V2P].astype(bf16)
    w2d = p_ref[OFF_W2:OFF_W2 + 4 * E, V2P:2 * V2P].astype(bf16)
    b2a = p_ref[OFF_B2:OFF_B2 + 1, 0:V2P]
    b2d = p_ref[OFF_B2:OFF_B2 + 1, V2P:2 * V2P]
    o_add = jnp.dot(h[:, :4 * E], w2a, preferred_element_type=f32) + b2a
    o_del = jnp.dot(h[:, 4 * E:], w2d, preferred_element_type=f32) + b2d

    out_ref[:, 0:V2P] = o_del
    out_ref[:, V2P:2 * V2P] = o_add
    out_ref[:, 2 * V2P:3 * V2P] = o_del


_call = pl.pallas_call(
    _body,
    grid=(B // TB,),
    in_specs=[
        pl.BlockSpec((TB, LD), lambda g: (g, 0)),
        pl.BlockSpec((TB, LP), lambda g: (g, 0)),
        pl.BlockSpec((R, W), lambda g: (0, 0)),
    ],
    out_specs=pl.BlockSpec((TB, 3 * V2P), lambda g: (g, 0)),
    out_shape=jax.ShapeDtypeStruct((B, 3 * V2P), jnp.float32),
    scratch_shapes=[pltpu.VMEM((V0P, TB), jnp.int16),
                    pltpu.VMEM((LD, TB), jnp.int32)],
    compiler_params=pltpu.CompilerParams(
        dimension_semantics=("parallel",)),
)


@jax.jit
def _forward(packed, diag_idx, prod_idx):
    raw = _call(jnp.asarray(diag_idx, jnp.int32),
                jnp.asarray(prod_idx, jnp.int32), packed)
    out = raw.reshape(B, 3, V2P)[:, :, :V2]
    return jnp.transpose(out, (0, 2, 1))


def kernel(packed, diag_idx, prod_idx):
    return _forward(packed, diag_idx, prod_idx)


# speedup vs baseline: 1.0179x; 1.0179x over previous
import jax
import jax.numpy as jnp
from jax import lax
from jax.experimental import pallas as pl
from jax.experimental.pallas import tpu as pltpu

V0, V1, V2 = 3584, 1536, 512
E = 128
B = 1024
LD, LP = 32, 16

V0P, V1P, V2P = 3584, 1536, 512
W = 1024
R = 6040
OFF_EMB0 = 0
OFF_EMB1 = 3584
OFF_WH = 5120
OFF_BH = 5376
OFF_W1 = 5384
OFF_B1 = 5512
OFF_W2 = 5520
OFF_B2 = 6032

BB_H = 0
BB_1 = E
BB_2A = E + 8 * E
BB_2D = BB_2A + V2P
BROWS = BB_2D + V2P

TB = 256
CONTRACT0 = (((0,), (0,)), ((), ()))


def _body(didx_ref, pidx_ref, p_ref, bias_ref, out_ref, cnt_ref, idxt_ref):
    f32 = jnp.float32
    bf16 = jnp.bfloat16
    i16 = jnp.int16

    def bag(idx_ref, ntaps, vocab_p, emb_off):
        idxt_ref[:ntaps, :] = jnp.transpose(idx_ref[...], (1, 0))
        row = lax.broadcasted_iota(i16, (vocab_p, TB), 0)
        cnt_ref[:vocab_p, :] = jnp.zeros((vocab_p, TB), i16)

        def tap8(i, c):
            v8 = idxt_ref[pl.ds(i * 8, 8), :].astype(i16)
            m = (row == v8[0:1, :]).astype(i16)
            for j in range(1, 8):
                m = m + (row == v8[j:j + 1, :]).astype(i16)
            cnt_ref[:vocab_p, :] = cnt_ref[:vocab_p, :] + m
            return c

        lax.fori_loop(0, ntaps // 8, tap8, 0)
        emb = p_ref[emb_off:emb_off + vocab_p, :E].astype(bf16)
        return lax.dot_general(emb, cnt_ref[:vocab_p, :].astype(bf16),
                               CONTRACT0, preferred_element_type=f32)

    dsumT = bag(didx_ref, LD, V0P, OFF_EMB0)
    psumT = bag(pidx_ref, LP, V1P, OFF_EMB1)
    hrT = jnp.concatenate([dsumT, psumT], axis=0).astype(bf16)

    wh = p_ref[OFF_WH:OFF_WH + 2 * E, :E].astype(bf16)
    repT = (lax.dot_general(wh, hrT, CONTRACT0,
                            preferred_element_type=f32)
            + bias_ref[BB_H:BB_H + E, :])

    w1 = p_ref[OFF_W1:OFF_W1 + E, :8 * E].astype(bf16)
    hT = jnp.maximum(
        lax.dot_general(w1, repT.astype(bf16), CONTRACT0,
                        preferred_element_type=f32)
        + bias_ref[BB_1:BB_1 + 8 * E, :], 0.0).astype(bf16)

    w2a = p_ref[OFF_W2:OFF_W2 + 4 * E, 0:V2P].astype(bf16)
    w2d = p_ref[OFF_W2:OFF_W2 + 4 * E, V2P:2 * V2P].astype(bf16)
    o_addT = (lax.dot_general(w2a, hT[:4 * E, :], CONTRACT0,
                              preferred_element_type=f32)
              + bias_ref[BB_2A:BB_2A + V2P, :])
    o_delT = (lax.dot_general(w2d, hT[4 * E:, :], CONTRACT0,
                              preferred_element_type=f32)
              + bias_ref[BB_2D:BB_2D + V2P, :])

    out_ref[0:V2P, :] = o_delT
    out_ref[V2P:2 * V2P, :] = o_addT
    out_ref[2 * V2P:3 * V2P, :] = o_delT


_call = pl.pallas_call(
    _body,
    grid=(B // TB,),
    in_specs=[
        pl.BlockSpec((TB, LD), lambda g: (g, 0)),
        pl.BlockSpec((TB, LP), lambda g: (g, 0)),
        pl.BlockSpec((R, W), lambda g: (0, 0)),
        pl.BlockSpec((BROWS, TB), lambda g: (0, 0)),
    ],
    out_specs=pl.BlockSpec((3 * V2P, TB), lambda g: (0, g)),
    out_shape=jax.ShapeDtypeStruct((3 * V2P, B), jnp.float32),
    scratch_shapes=[pltpu.VMEM((V0P, TB), jnp.int16),
                    pltpu.VMEM((LD, TB), jnp.int32)],
    compiler_params=pltpu.CompilerParams(
        dimension_semantics=("parallel",)),
)


@jax.jit
def _forward(packed, diag_idx, prod_idx):
    bias = jnp.concatenate([
        packed[OFF_BH, :E],
        packed[OFF_B1, :8 * E],
        packed[OFF_B2, 0:V2P],
        packed[OFF_B2, V2P:2 * V2P],
    ])
    biasT = jnp.broadcast_to(bias[:, None], (BROWS, TB))
    raw = _call(jnp.asarray(diag_idx, jnp.int32),
                jnp.asarray(prod_idx, jnp.int32), packed, biasT)
    out = raw.reshape(3, V2P, B)[:, :V2, :]
    return jnp.transpose(out, (2, 1, 0))


def kernel(packed, diag_idx, prod_idx):
    return _forward(packed, diag_idx, prod_idx)


# speedup vs baseline: 1.1762x; 1.1555x over previous
import jax
import jax.numpy as jnp
from jax import lax
from jax.experimental import pallas as pl
from jax.experimental.pallas import tpu as pltpu

V0, V1, V2 = 3584, 1536, 512
E = 128
B = 1024
LD, LP = 32, 16

V0P, V1P, V2P = 3584, 1536, 512
W = 1024
R = 6040
OFF_EMB0 = 0
OFF_EMB1 = 3584
OFF_WH = 5120
OFF_BH = 5376
OFF_W1 = 5384
OFF_B1 = 5512
OFF_W2 = 5520
OFF_B2 = 6032
MLP_ROWS = R - OFF_WH
M_WH = OFF_WH - OFF_WH
M_W1 = OFF_W1 - OFF_WH
M_W2 = OFF_W2 - OFF_WH

BB_H = 0
BB_1 = E
BB_2A = E + 8 * E
BB_2D = BB_2A + V2P
BROWS = BB_2D + V2P

TB = 256
CONTRACT0 = (((0,), (0,)), ((), ()))


def _body(didx_ref, pidx_ref, p_hbm, bias_ref, out_ref,
          emb0_s, emb1_s, mlp_s, dcnt_ref, pcnt_ref, sem):
    f32 = jnp.float32
    bf16 = jnp.bfloat16
    i16 = jnp.int16

    cp0 = pltpu.make_async_copy(
        p_hbm.at[pl.ds(OFF_EMB0, V0P), pl.ds(0, E)], emb0_s, sem.at[0])
    cp1 = pltpu.make_async_copy(
        p_hbm.at[pl.ds(OFF_EMB1, V1P), pl.ds(0, E)], emb1_s, sem.at[1])
    cp2 = pltpu.make_async_copy(
        p_hbm.at[pl.ds(OFF_WH, MLP_ROWS), pl.ds(0, W)], mlp_s, sem.at[2])
    cp0.start()
    cp1.start()
    cp2.start()

    def bag(idx_ref, cnt_ref, ntaps, vocab_p):
        row = lax.broadcasted_iota(i16, (vocab_p, TB), 0)
        cnt_ref[...] = jnp.zeros((vocab_p, TB), i16)

        def tap8(i, c):
            v8 = idx_ref[pl.ds(i * 8, 8), :].astype(i16)
            m = (row == v8[0:1, :]).astype(i16)
            for j in range(1, 8):
                m = m + (row == v8[j:j + 1, :]).astype(i16)
            cnt_ref[...] = cnt_ref[...] + m
            return c

        lax.fori_loop(0, ntaps // 8, tap8, 0)

    bag(didx_ref, dcnt_ref, LD, V0P)
    cp0.wait()
    dsumT = lax.dot_general(emb0_s[...].astype(bf16),
                            dcnt_ref[...].astype(bf16),
                            CONTRACT0, preferred_element_type=f32)
    bag(pidx_ref, pcnt_ref, LP, V1P)
    cp1.wait()
    psumT = lax.dot_general(emb1_s[...].astype(bf16),
                            pcnt_ref[...].astype(bf16),
                            CONTRACT0, preferred_element_type=f32)
    hrT = jnp.concatenate([dsumT, psumT], axis=0).astype(bf16)

    cp2.wait()
    wh = mlp_s[M_WH:M_WH + 2 * E, :E].astype(bf16)
    repT = (lax.dot_general(wh, hrT, CONTRACT0,
                            preferred_element_type=f32)
            + bias_ref[BB_H:BB_H + E, :])

    w1 = mlp_s[M_W1:M_W1 + E, :8 * E].astype(bf16)
    hT = jnp.maximum(
        lax.dot_general(w1, repT.astype(bf16), CONTRACT0,
                        preferred_element_type=f32)
        + bias_ref[BB_1:BB_1 + 8 * E, :], 0.0).astype(bf16)

    w2a = mlp_s[M_W2:M_W2 + 4 * E, 0:V2P].astype(bf16)
    w2d = mlp_s[M_W2:M_W2 + 4 * E, V2P:2 * V2P].astype(bf16)
    o_addT = (lax.dot_general(w2a, hT[:4 * E, :], CONTRACT0,
                              preferred_element_type=f32)
              + bias_ref[BB_2A:BB_2A + V2P, :])
    o_delT = (lax.dot_general(w2d, hT[4 * E:, :], CONTRACT0,
                              preferred_element_type=f32)
              + bias_ref[BB_2D:BB_2D + V2P, :])

    out_ref[0:V2P, :] = o_delT
    out_ref[V2P:2 * V2P, :] = o_addT
    out_ref[2 * V2P:3 * V2P, :] = o_delT


_call = pl.pallas_call(
    _body,
    grid=(B // TB,),
    in_specs=[
        pl.BlockSpec((LD, TB), lambda g: (0, g)),
        pl.BlockSpec((LP, TB), lambda g: (0, g)),
        pl.BlockSpec(memory_space=pl.ANY),
        pl.BlockSpec((BROWS, TB), lambda g: (0, 0)),
    ],
    out_specs=pl.BlockSpec((3 * V2P, TB), lambda g: (0, g)),
    out_shape=jax.ShapeDtypeStruct((3 * V2P, B), jnp.float32),
    scratch_shapes=[pltpu.VMEM((V0P, E), jnp.float32),
                    pltpu.VMEM((V1P, E), jnp.float32),
                    pltpu.VMEM((MLP_ROWS, W), jnp.float32),
                    pltpu.VMEM((V0P, TB), jnp.int16),
                    pltpu.VMEM((V1P, TB), jnp.int16),
                    pltpu.SemaphoreType.DMA((3,))],
    compiler_params=pltpu.CompilerParams(
        dimension_semantics=("parallel",)),
)


@jax.jit
def _forward(packed, diag_idx, prod_idx):
    diagT = jnp.asarray(diag_idx, jnp.int32).T
    prodT = jnp.asarray(prod_idx, jnp.int32).T
    bias = jnp.concatenate([
        packed[OFF_BH, :E],
        packed[OFF_B1, :8 * E],
        packed[OFF_B2, 0:V2P],
        packed[OFF_B2, V2P:2 * V2P],
    ])
    biasT = jnp.broadcast_to(bias[:, None], (BROWS, TB))
    raw = _call(diagT, prodT, packed, biasT)
    out = raw.reshape(3, V2P, B)[:, :V2, :]
    return jnp.transpose(out, (2, 1, 0))


def kernel(packed, diag_idx, prod_idx):
    return _forward(packed, diag_idx, prod_idx)
